# baseline (speedup 1.0000x reference)
import functools

import jax
import jax.numpy as jnp
from jax import lax
from jax.experimental import pallas as pl
from jax.experimental.pallas import tpu as pltpu
from jax.experimental.pallas import tpu_sc as plsc

L = 16


@functools.lru_cache(maxsize=None)
def _build_sc_gather(nrows, d, f, chunk_rows):
    info = plsc.get_sparse_core_info()
    nc, ns = info.num_cores, info.num_subcores
    nw = nc * ns
    assert nrows % (nw * chunk_rows) == 0
    rows_per_worker = nrows // nw
    n_chunks = rows_per_worker // chunk_rows
    n_groups = f // L

    mesh = plsc.VectorSubcoreMesh(core_axis_name="c", subcore_axis_name="s")

    @functools.partial(
        pl.kernel,
        out_type=jax.ShapeDtypeStruct((nrows * f,), jnp.float32),
        mesh=mesh,
        compiler_params=pltpu.CompilerParams(needs_layout_passes=False),
        scratch_types=[
            pltpu.VMEM((f,), jnp.int32),
            pltpu.VMEM((chunk_rows * d,), jnp.float32),
            pltpu.VMEM((chunk_rows * f,), jnp.float32),
            pltpu.SemaphoreType.DMA,
            pltpu.SemaphoreType.DMA,
        ],
    )
    def sc_gather(x_hbm, idx_hbm, out_hbm, idx_v, in_b, out_b, sem_in, sem_out):
        wid = lax.axis_index("s") * nc + lax.axis_index("c")
        base = wid * rows_per_worker
        pltpu.sync_copy(idx_hbm, idx_v)
        col_idx = [idx_v[pl.ds(g * L, L)] for g in range(n_groups)]

        def body(c, carry):
            r0 = base + c * chunk_rows
            cp_in = pltpu.make_async_copy(
                x_hbm.at[pl.ds(r0 * d, chunk_rows * d)], in_b, sem_in)
            cp_in.start()
            cp_in.wait()
            for i in range(chunk_rows):
                for g in range(n_groups):
                    vals = plsc.load_gather(in_b, [col_idx[g] + (i * d)])
                    out_b[pl.ds(i * f + g * L, L)] = vals
            cp_out = pltpu.make_async_copy(
                out_b, out_hbm.at[pl.ds(r0 * f, chunk_rows * f)], sem_out)
            cp_out.start()
            cp_out.wait()
            return carry

        lax.fori_loop(0, n_chunks, body, 0)

    return sc_gather


def kernel(X, feature_indices):
    b, s, d = X.shape
    f = feature_indices.shape[0]
    nrows = b * s
    x_flat = X.reshape(nrows * d)
    out_flat = _build_sc_gather(nrows, d, f, 16)(x_flat, feature_indices)
    return out_flat.reshape(b, s, f)


# speedup vs baseline: 1.1756x; 1.1756x over previous
import functools

import jax
import jax.numpy as jnp
from jax import lax
from jax.experimental import pallas as pl
from jax.experimental.pallas import tpu as pltpu
from jax.experimental.pallas import tpu_sc as plsc

L = 16


@functools.lru_cache(maxsize=None)
def _build_sc_gather(nrows, d, f, chunk_rows):
    info = plsc.get_sparse_core_info()
    nc, ns = info.num_cores, info.num_subcores
    nw = nc * ns
    assert nrows % (nw * chunk_rows) == 0
    rows_per_worker = nrows // nw
    n_chunks = rows_per_worker // chunk_rows
    n_groups = f // L

    mesh = plsc.VectorSubcoreMesh(core_axis_name="c", subcore_axis_name="s")

    assert n_chunks % 2 == 0
    n_pairs = n_chunks // 2

    @functools.partial(
        pl.kernel,
        out_type=jax.ShapeDtypeStruct((nrows * f,), jnp.float32),
        mesh=mesh,
        compiler_params=pltpu.CompilerParams(needs_layout_passes=False),
        scratch_types=[
            pltpu.VMEM((f,), jnp.int32),
            pltpu.VMEM((chunk_rows * d,), jnp.float32),
            pltpu.VMEM((chunk_rows * d,), jnp.float32),
            pltpu.VMEM((chunk_rows * f,), jnp.float32),
            pltpu.VMEM((chunk_rows * f,), jnp.float32),
            pltpu.SemaphoreType.DMA,
            pltpu.SemaphoreType.DMA,
            pltpu.SemaphoreType.DMA,
            pltpu.SemaphoreType.DMA,
        ],
    )
    def sc_gather(x_hbm, idx_hbm, out_hbm, idx_v, in0, in1, ob0, ob1,
                  si0, si1, so0, so1):
        wid = lax.axis_index("s") * nc + lax.axis_index("c")
        base = wid * rows_per_worker
        pltpu.sync_copy(idx_hbm, idx_v)
        col_idx = [idx_v[pl.ds(g * L, L)] for g in range(n_groups)]

        def cp_in(c, buf, sem):
            return pltpu.make_async_copy(
                x_hbm.at[pl.ds((base + c * chunk_rows) * d, chunk_rows * d)],
                buf, sem)

        def cp_out(c, buf, sem):
            return pltpu.make_async_copy(
                buf,
                out_hbm.at[pl.ds((base + c * chunk_rows) * f, chunk_rows * f)],
                sem)

        def subsample(in_b, out_b):
            for i in range(chunk_rows):
                for g in range(n_groups):
                    vals = plsc.load_gather(in_b, [col_idx[g] + (i * d)])
                    out_b[pl.ds(i * f + g * L, L)] = vals

        cp_in(0, in0, si0).start()
        cp_in(1, in1, si1).start()

        def body(k, carry):
            c0 = 2 * k
            cp_in(c0, in0, si0).wait()

            @pl.when(k > 0)
            def _():
                cp_out(c0 - 2, ob0, so0).wait()

            subsample(in0, ob0)
            cp_out(c0, ob0, so0).start()

            @pl.when(k < n_pairs - 1)
            def _():
                cp_in(c0 + 2, in0, si0).start()

            cp_in(c0 + 1, in1, si1).wait()

            @pl.when(k > 0)
            def _():
                cp_out(c0 - 1, ob1, so1).wait()

            subsample(in1, ob1)
            cp_out(c0 + 1, ob1, so1).start()

            @pl.when(k < n_pairs - 1)
            def _():
                cp_in(c0 + 3, in1, si1).start()

            return carry

        lax.fori_loop(0, n_pairs, body, 0)
        cp_out(n_chunks - 2, ob0, so0).wait()
        cp_out(n_chunks - 1, ob1, so1).wait()

    return sc_gather


def kernel(X, feature_indices):
    b, s, d = X.shape
    f = feature_indices.shape[0]
    nrows = b * s
    x_flat = X.reshape(nrows * d)
    out_flat = _build_sc_gather(nrows, d, f, 16)(x_flat, feature_indices)
    return out_flat.reshape(b, s, f)


# speedup vs baseline: 2.2507x; 1.9144x over previous
import functools

import jax
import jax.numpy as jnp
from jax import lax
from jax.experimental import pallas as pl
from jax.experimental.pallas import tpu as pltpu
from jax.experimental.pallas import tpu_sc as plsc

L = 16


@functools.lru_cache(maxsize=None)
def _build_sc_gather(nrows, d, f, chunk_rows):
    info = plsc.get_sparse_core_info()
    nc, ns = info.num_cores, info.num_subcores
    nw = nc * ns
    assert nrows % (nw * chunk_rows) == 0
    rows_per_worker = nrows // nw
    n_chunks = rows_per_worker // chunk_rows
    n_groups = f // L

    mesh = plsc.VectorSubcoreMesh(core_axis_name="c", subcore_axis_name="s")

    assert n_chunks % 2 == 0
    n_pairs = n_chunks // 2

    @functools.partial(
        pl.kernel,
        out_type=jax.ShapeDtypeStruct((nrows, f), jnp.float32),
        mesh=mesh,
        compiler_params=pltpu.CompilerParams(needs_layout_passes=False),
        scratch_types=[
            pltpu.VMEM((f,), jnp.int32),
            pltpu.VMEM((chunk_rows, d), jnp.float32),
            pltpu.VMEM((chunk_rows, d), jnp.float32),
            pltpu.VMEM((chunk_rows, f), jnp.float32),
            pltpu.VMEM((chunk_rows, f), jnp.float32),
            pltpu.SemaphoreType.DMA,
            pltpu.SemaphoreType.DMA,
            pltpu.SemaphoreType.DMA,
            pltpu.SemaphoreType.DMA,
        ],
    )
    def sc_gather(x_hbm, idx_hbm, out_hbm, idx_v, in0, in1, ob0, ob1,
                  si0, si1, so0, so1):
        wid = lax.axis_index("s") * nc + lax.axis_index("c")
        base = wid * rows_per_worker
        pltpu.sync_copy(idx_hbm, idx_v)
        col_idx = [idx_v[pl.ds(g * L, L)] for g in range(n_groups)]

        def cp_in(c, buf, sem):
            return pltpu.make_async_copy(
                x_hbm.at[pl.ds(base + c * chunk_rows, chunk_rows)], buf, sem)

        def cp_out(c, buf, sem):
            return pltpu.make_async_copy(
                buf, out_hbm.at[pl.ds(base + c * chunk_rows, chunk_rows)], sem)

        def subsample(in_b, out_b):
            for i in range(chunk_rows):
                row = jnp.full((L,), i, jnp.int32)
                for g in range(n_groups):
                    vals = plsc.load_gather(in_b, [row, col_idx[g]])
                    out_b[i, pl.ds(g * L, L)] = vals

        cp_in(0, in0, si0).start()
        cp_in(1, in1, si1).start()

        def body(k, carry):
            c0 = 2 * k
            cp_in(c0, in0, si0).wait()

            @pl.when(k > 0)
            def _():
                cp_out(c0 - 2, ob0, so0).wait()

            subsample(in0, ob0)
            cp_out(c0, ob0, so0).start()

            @pl.when(k < n_pairs - 1)
            def _():
                cp_in(c0 + 2, in0, si0).start()

            cp_in(c0 + 1, in1, si1).wait()

            @pl.when(k > 0)
            def _():
                cp_out(c0 - 1, ob1, so1).wait()

            subsample(in1, ob1)
            cp_out(c0 + 1, ob1, so1).start()

            @pl.when(k < n_pairs - 1)
            def _():
                cp_in(c0 + 3, in1, si1).start()

            return carry

        lax.fori_loop(0, n_pairs, body, 0)
        cp_out(n_chunks - 2, ob0, so0).wait()
        cp_out(n_chunks - 1, ob1, so1).wait()

    return sc_gather


def kernel(X, feature_indices):
    b, s, d = X.shape
    f = feature_indices.shape[0]
    nrows = b * s
    x2d = X.reshape(nrows, d)
    out2d = _build_sc_gather(nrows, d, f, 16)(x2d, feature_indices)
    return out2d.reshape(b, s, f)


# speedup vs baseline: 4.2907x; 1.9064x over previous
import functools

import jax
import jax.numpy as jnp
from jax import lax
from jax.experimental import pallas as pl
from jax.experimental.pallas import tpu as pltpu
from jax.experimental.pallas import tpu_sc as plsc

L = 16


@functools.lru_cache(maxsize=None)
def _build_sc_gather(nrows, d, f, chunk_rows):
    info = plsc.get_sparse_core_info()
    nc, ns = info.num_cores, info.num_subcores
    nw = nc * ns
    assert nrows % (nw * chunk_rows) == 0
    rows_per_worker = nrows // nw
    n_chunks = rows_per_worker // chunk_rows
    n_groups = f // L

    mesh = plsc.VectorSubcoreMesh(core_axis_name="c", subcore_axis_name="s")

    assert n_chunks % 2 == 0
    n_pairs = n_chunks // 2

    @functools.partial(
        pl.kernel,
        out_type=jax.ShapeDtypeStruct((nrows, f), jnp.float32),
        mesh=mesh,
        compiler_params=pltpu.CompilerParams(needs_layout_passes=False),
        scratch_types=[
            pltpu.VMEM((f,), jnp.int32),
            pltpu.VMEM((chunk_rows, d), jnp.float32),
            pltpu.VMEM((chunk_rows, d), jnp.float32),
            pltpu.VMEM((chunk_rows, f), jnp.float32),
            pltpu.VMEM((chunk_rows, f), jnp.float32),
            pltpu.SemaphoreType.DMA,
            pltpu.SemaphoreType.DMA,
            pltpu.SemaphoreType.DMA,
            pltpu.SemaphoreType.DMA,
        ],
    )
    def sc_gather(x_hbm, idx_hbm, out_hbm, idx_v, in0, in1, ob0, ob1,
                  si0, si1, so0, so1):
        wid = lax.axis_index("s") * nc + lax.axis_index("c")
        base = wid * rows_per_worker
        pltpu.sync_copy(idx_hbm, idx_v)
        col_idx = [idx_v[pl.ds(g * L, L)] for g in range(n_groups)]

        def cp_in(c, buf, sem):
            return pltpu.make_async_copy(
                x_hbm.at[pl.ds(base + c * chunk_rows, chunk_rows)], buf, sem)

        def cp_out(c, buf, sem):
            return pltpu.make_async_copy(
                buf, out_hbm.at[pl.ds(base + c * chunk_rows, chunk_rows)], sem)

        def subsample(in_b, out_b):
            for i in range(chunk_rows):
                row = jnp.full((L,), i, jnp.int32)
                for g in range(n_groups):
                    vals = plsc.load_gather(in_b, [row, col_idx[g]])
                    out_b[i, pl.ds(g * L, L)] = vals

        cp_in(0, in0, si0).start()
        cp_in(1, in1, si1).start()

        def body(k, carry):
            c0 = 2 * k
            cp_in(c0, in0, si0).wait()

            @pl.when(k > 0)
            def _():
                cp_out(c0 - 2, ob0, so0).wait()

            subsample(in0, ob0)
            cp_out(c0, ob0, so0).start()

            @pl.when(k < n_pairs - 1)
            def _():
                cp_in(c0 + 2, in0, si0).start()

            cp_in(c0 + 1, in1, si1).wait()

            @pl.when(k > 0)
            def _():
                cp_out(c0 - 1, ob1, so1).wait()

            subsample(in1, ob1)
            cp_out(c0 + 1, ob1, so1).start()

            @pl.when(k < n_pairs - 1)
            def _():
                cp_in(c0 + 3, in1, si1).start()

            return carry

        lax.fori_loop(0, n_pairs, body, 0)
        cp_out(n_chunks - 2, ob0, so0).wait()
        cp_out(n_chunks - 1, ob1, so1).wait()

    return sc_gather


@functools.lru_cache(maxsize=None)
def _build_tc_gather(nrows_tc, d, f, br, row_off):
    def body(idx_ref, x_ref, o_ref, oh_ref):
        @pl.when(pl.program_id(0) == 0)
        def _():
            di = lax.broadcasted_iota(jnp.int32, (d, f), 0)
            oh_ref[...] = jnp.where(di == idx_ref[0, :][None, :], 1.0, 0.0)

        o_ref[...] = jnp.dot(x_ref[...], oh_ref[...],
                             preferred_element_type=jnp.float32)

    return pl.pallas_call(
        body,
        grid=(nrows_tc // br,),
        in_specs=[
            pl.BlockSpec((8, f), lambda i: (0, 0)),
            pl.BlockSpec((br, d), lambda i: (i + row_off // br, 0)),
        ],
        out_specs=pl.BlockSpec((br, f), lambda i: (i, 0)),
        out_shape=jax.ShapeDtypeStruct((nrows_tc, f), jnp.float32),
        scratch_shapes=[pltpu.VMEM((d, f), jnp.float32)],
    )


def kernel(X, feature_indices):
    b, s, d = X.shape
    f = feature_indices.shape[0]
    nrows = b * s
    x2d = X.reshape(nrows, d)
    idx8 = jnp.broadcast_to(feature_indices, (8, f))
    out2d = _build_tc_gather(nrows, d, f, 512, 0)(idx8, x2d)
    return out2d.reshape(b, s, f)
